# baseline (speedup 1.0000x reference)
import functools

import jax
import jax.numpy as jnp
from jax import lax
from jax.experimental import pallas as pl
from jax.experimental.pallas import tpu as pltpu
from jax.experimental.pallas import tpu_sc as plsc

NUM_ENT = 1000000
NUM_REL = 1000
EMB_DIM = 64
BATCH = 16384

NC = 2
NS = 16
NW = NC * NS
B_PER_W = BATCH // NW
CHUNK = 64
NCHUNK = B_PER_W // CHUNK
L = 16
NG = EMB_DIM // L


def _mdist_kernel(e_hbm, r_hbm, idx_hbm, out_hbm, idx_v, rows_v, out_v, sem):
    wid = lax.axis_index("s") * NC + lax.axis_index("c")

    for k in range(7):
        pltpu.sync_copy(idx_hbm.at[k, wid], idx_v.at[k])

    iota = lax.broadcasted_iota(jnp.int32, (L,), 0)

    for ch in range(NCHUNK):
        copies = []
        for k in range(7):
            tbl = r_hbm if k == 0 else e_hbm
            copies.append(
                pltpu.async_copy(tbl.at[idx_v.at[k, ch]], rows_v.at[k], sem))
        for cp in copies:
            cp.wait()

        for g in range(CHUNK // L):

            def rbody(j, sums):
                b = g * L + j
                acc = None
                for gg in range(NG):
                    p = rows_v[0, b, pl.ds(gg * L, L)]
                    for k in range(1, 7):
                        p = p * rows_v[k, b, pl.ds(gg * L, L)]
                    acc = p if acc is None else acc + p
                s = jnp.sum(acc)
                return jnp.where(iota == j, s, sums)

            sums = lax.fori_loop(0, L, rbody, jnp.zeros((L,), jnp.float32))
            out_v[ch, pl.ds(g * L, L)] = sums

    pltpu.sync_copy(out_v, out_hbm.at[wid])


@jax.jit
def _mdist(idx_all, e_weight, r_weight):
    mesh = plsc.VectorSubcoreMesh(core_axis_name="c", subcore_axis_name="s")
    run = functools.partial(
        pl.kernel,
        mesh=mesh,
        compiler_params=pltpu.CompilerParams(
            needs_layout_passes=False, use_tc_tiling_on_sc=False),
        out_type=jax.ShapeDtypeStruct((NW, NCHUNK, CHUNK), jnp.float32),
        scratch_types=[
            pltpu.VMEM((7, NCHUNK, CHUNK), jnp.int32),
            pltpu.VMEM((7, CHUNK, EMB_DIM), jnp.float32),
            pltpu.VMEM((NCHUNK, CHUNK), jnp.float32),
            pltpu.SemaphoreType.DMA,
        ],
    )(_mdist_kernel)
    return run(e_weight, r_weight, idx_all)


def kernel(r_idx, e1_idx, e2_idx, e3_idx, e4_idx, e5_idx, e6_idx,
           E_weight, R_weight):
    idx_all = jnp.stack([
        r_idx, e1_idx, e2_idx, e3_idx, e4_idx, e5_idx, e6_idx,
    ]).astype(jnp.int32).reshape(7, NW, NCHUNK, CHUNK)
    out = _mdist(idx_all, E_weight, R_weight)
    return out.reshape(BATCH)
